# Initial kernel scaffold; baseline (speedup 1.0000x reference)
import jax
import jax.numpy as jnp
from jax.experimental import pallas as pl


def kernel(words, sent_id, heads_ids, emb, W1, b1, W2, b2, W3, b3, c1w, c1b, c2w, c2b):
    raise NotImplementedError("write your pallas kernel here")


# speedup vs baseline: 7.7756x; 7.7756x over previous
import jax
import jax.numpy as jnp
from jax.experimental import pallas as pl
from jax.experimental.pallas import tpu as pltpu

_W = 128
_WE = _W + 8
_NEG_INF = float("-inf")


def _dot(a, b):
    return jax.lax.dot_general(a, b, (((1,), (0,)), ((), ())),
                               preferred_element_type=jnp.float32)


def _tc_kernel(starts_ref, lens_ref, heads_ref, ml_ref,
               words_ref, emb_ref,
               w1h_ref, w1c_ref, w1e_ref, b1_ref,
               w2_ref, b2_ref, w3_ref, b3_ref,
               c1w_ref, c1b_ref, c2w_ref, c2b_ref,
               out_ref,
               win_ref, hrow_ref, sem_ref):
    i = pl.program_id(0)
    s = starts_ref[i]
    L = lens_ref[i]
    hd = heads_ref[i]
    ml = ml_ref[0]

    s8 = pl.multiple_of((s // 8) * 8, 8)
    d = s - s8
    hd8 = pl.multiple_of((hd // 8) * 8, 8)
    cp_win = pltpu.make_async_copy(
        words_ref.at[pl.ds(s8, _WE), :], win_ref, sem_ref.at[0])
    cp_hrow = pltpu.make_async_copy(
        words_ref.at[pl.ds(hd8, 8), :], hrow_ref, sem_ref.at[1])
    cp_win.start()
    cp_hrow.start()
    cp_win.wait()
    cp_hrow.wait()

    t = jax.lax.broadcasted_iota(jnp.int32, (_WE, 1), 0)

    t8 = jax.lax.broadcasted_iota(jnp.int32, (8, 1), 0)
    hrow = jnp.sum(jnp.where(t8 == hd - hd8, hrow_ref[...], 0.0),
                   axis=0, keepdims=True)

    rel = hd - (s8 + t)
    eid = rel + 63
    eid = jnp.where((eid < 0) | (eid > 126), 127, eid)
    lane = jax.lax.broadcasted_iota(jnp.int32, (_WE, 128), 1)
    onehot = (lane == eid).astype(jnp.float32)
    embrows = _dot(onehot, emb_ref[...])

    in_span = (t >= d) & (t < d + L)
    in_len = (t >= d) & (t < d + ml)

    pre = (_dot(win_ref[...], w1c_ref[...])
           + _dot(embrows, w1e_ref[...])
           + _dot(hrow, w1h_ref[...]))
    pre = jnp.where(in_span, pre, 0.0)
    h = jnp.maximum(pre + b1_ref[...], 0.0)
    h = jnp.maximum(_dot(h, w2_ref[...]) + b2_ref[...], 0.0)
    h = _dot(h, w3_ref[...]) + b3_ref[...]
    hm = jnp.where(in_len, h, 0.0)

    z64 = jnp.zeros((1, 64), jnp.float32)
    hprev = jnp.concatenate([z64, hm[:-1]], axis=0)
    hnext = jnp.concatenate([hm[1:], z64], axis=0)
    g = (_dot(hprev, c1w_ref[0]) + _dot(hm, c1w_ref[1])
         + _dot(hnext, c1w_ref[2]) + c1b_ref[...])
    gm = jnp.where(in_len, g, 0.0)

    z4 = jnp.zeros((1, 4), jnp.float32)
    gprev = jnp.concatenate([z4, gm[:-1]], axis=0)
    gnext = jnp.concatenate([gm[1:], z4], axis=0)
    r = (_dot(gprev, c2w_ref[0]) + _dot(gm, c2w_ref[1])
         + _dot(gnext, c2w_ref[2]) + c2b_ref[...])

    t2 = jax.lax.broadcasted_iota(jnp.int32, (_WE, 2), 0)
    lane2 = jax.lax.broadcasted_iota(jnp.int32, (_WE, 2), 1)
    rel2 = hd - (s8 + t2)
    sgn = 1 - 2 * lane2
    valid = ((rel2 * sgn) >= 0) & (t2 >= d) & (t2 < d + L)
    out_ref[0] = jnp.where(valid, r, _NEG_INF)


def kernel(words, sent_id, heads_ids, emb,
           W1, b1, W2, b2, W3, b3, c1w, c1b, c2w, c2b):
    n_words, in_size = words.shape
    n_heads = heads_ids.shape[0]

    heads = heads_ids.astype(jnp.int32)
    sid_h = jnp.take(sent_id, heads, axis=0)
    starts = jnp.searchsorted(sent_id, sid_h, side="left").astype(jnp.int32)
    ends = jnp.searchsorted(sent_id, sid_h, side="right").astype(jnp.int32)
    lens = ends - starts
    ml = jnp.max(lens)[None].astype(jnp.int32)

    words_pad = jnp.concatenate(
        [words, jnp.zeros((_WE, in_size), words.dtype)], axis=0)

    w1h = W1[:, :in_size].T
    w1c = W1[:, in_size:2 * in_size].T
    w1e = W1[:, 2 * in_size:].T
    c1wT = jnp.transpose(c1w, (2, 1, 0))
    c2wT = jnp.transpose(c2w, (2, 1, 0))

    grid_spec = pltpu.PrefetchScalarGridSpec(
        num_scalar_prefetch=4,
        grid=(n_heads,),
        in_specs=[
            pl.BlockSpec(memory_space=pl.ANY),
            pl.BlockSpec(emb.shape, lambda i, *_: (0, 0)),
            pl.BlockSpec(w1h.shape, lambda i, *_: (0, 0)),
            pl.BlockSpec(w1c.shape, lambda i, *_: (0, 0)),
            pl.BlockSpec(w1e.shape, lambda i, *_: (0, 0)),
            pl.BlockSpec((1, b1.shape[0]), lambda i, *_: (0, 0)),
            pl.BlockSpec((W2.shape[1], W2.shape[0]), lambda i, *_: (0, 0)),
            pl.BlockSpec((1, b2.shape[0]), lambda i, *_: (0, 0)),
            pl.BlockSpec((W3.shape[1], W3.shape[0]), lambda i, *_: (0, 0)),
            pl.BlockSpec((1, b3.shape[0]), lambda i, *_: (0, 0)),
            pl.BlockSpec(c1wT.shape, lambda i, *_: (0, 0, 0)),
            pl.BlockSpec((1, c1b.shape[0]), lambda i, *_: (0, 0)),
            pl.BlockSpec(c2wT.shape, lambda i, *_: (0, 0, 0)),
            pl.BlockSpec((1, c2b.shape[0]), lambda i, *_: (0, 0)),
        ],
        out_specs=pl.BlockSpec((1, _WE, 2), lambda i, *_: (i, 0, 0)),
        scratch_shapes=[
            pltpu.VMEM((_WE, in_size), jnp.float32),
            pltpu.VMEM((8, in_size), jnp.float32),
            pltpu.SemaphoreType.DMA((2,)),
        ],
    )

    packed = pl.pallas_call(
        _tc_kernel,
        grid_spec=grid_spec,
        out_shape=jax.ShapeDtypeStruct((n_heads, _WE, 2), jnp.float32),
    )(starts, lens, heads, ml,
      words_pad, emb,
      w1h, w1c, w1e, b1[None, :],
      W2.T, b2[None, :], W3.T, b3[None, :],
      c1wT, c1b[None, :], c2wT, c2b[None, :])

    cols = ((starts // 8) * 8)[:, None] + jnp.arange(_WE, dtype=jnp.int32)[None, :]
    out = jnp.full((n_heads, n_words, 2), _NEG_INF, jnp.float32)
    out = out.at[jnp.arange(n_heads)[:, None], cols].set(packed, mode="drop")
    return out


# speedup vs baseline: 22.8863x; 2.9433x over previous
import jax
import jax.numpy as jnp
from jax.experimental import pallas as pl
from jax.experimental.pallas import tpu as pltpu

_W = 128
_WE = _W + 8
_NEG_INF = float("-inf")


def _dot(a, b):
    return jax.lax.dot_general(a, b, (((1,), (0,)), ((), ())),
                               preferred_element_type=jnp.float32)


def _tc_kernel(starts_ref, lens_ref, heads_ref, ml_ref,
               words_ref, emb_ref,
               w1h_ref, w1c_ref, w1e_ref, b1_ref,
               w2_ref, b2_ref, w3_ref, b3_ref,
               c1w_ref, c1b_ref, c2w_ref, c2b_ref, init_ref,
               out_ref,
               owin_ref, sem_ref):
    i = pl.program_id(0)
    s = starts_ref[i]
    L = lens_ref[i]
    hd = heads_ref[i]
    ml = ml_ref[0]

    s8 = pl.multiple_of((s // 8) * 8, 8)
    d = s - s8
    hd8 = pl.multiple_of((hd // 8) * 8, 8)
    win = words_ref[pl.ds(s8, _WE), :]
    hblk = words_ref[pl.ds(hd8, 8), :]

    t = jax.lax.broadcasted_iota(jnp.int32, (_WE, 1), 0)

    t8 = jax.lax.broadcasted_iota(jnp.int32, (8, 1), 0)
    hrow = jnp.sum(jnp.where(t8 == hd - hd8, hblk, 0.0),
                   axis=0, keepdims=True)

    rel = hd - (s8 + t)
    eid = rel + 63
    eid = jnp.where((eid < 0) | (eid > 126), 127, eid)
    lane = jax.lax.broadcasted_iota(jnp.int32, (_WE, 128), 1)
    onehot = (lane == eid).astype(jnp.float32)
    embrows = _dot(onehot, emb_ref[...])

    in_span = (t >= d) & (t < d + L)
    in_len = (t >= d) & (t < d + ml)

    pre = (_dot(win, w1c_ref[...])
           + _dot(embrows, w1e_ref[...])
           + _dot(hrow, w1h_ref[...]))
    pre = jnp.where(in_span, pre, 0.0)
    h = jnp.maximum(pre + b1_ref[...], 0.0)
    h = jnp.maximum(_dot(h, w2_ref[...]) + b2_ref[...], 0.0)
    h = _dot(h, w3_ref[...]) + b3_ref[...]
    hm = jnp.where(in_len, h, 0.0)

    z64 = jnp.zeros((1, 64), jnp.float32)
    hprev = jnp.concatenate([z64, hm[:-1]], axis=0)
    hnext = jnp.concatenate([hm[1:], z64], axis=0)
    g = (_dot(hprev, c1w_ref[0]) + _dot(hm, c1w_ref[1])
         + _dot(hnext, c1w_ref[2]) + c1b_ref[...])
    gm = jnp.where(in_len, g, 0.0)

    z4 = jnp.zeros((1, 4), jnp.float32)
    gprev = jnp.concatenate([z4, gm[:-1]], axis=0)
    gnext = jnp.concatenate([gm[1:], z4], axis=0)
    r = (_dot(gprev, c2w_ref[0]) + _dot(gm, c2w_ref[1])
         + _dot(gnext, c2w_ref[2]) + c2b_ref[...])

    t2 = jax.lax.broadcasted_iota(jnp.int32, (_WE, 2), 0)
    lane2 = jax.lax.broadcasted_iota(jnp.int32, (_WE, 2), 1)
    rel2 = hd - (s8 + t2)
    sgn = 1 - 2 * lane2
    valid = ((rel2 * sgn) >= 0) & (t2 >= d) & (t2 < d + L)
    owin_ref[...] = jnp.where(valid, r, _NEG_INF)

    cp_out = pltpu.make_async_copy(
        owin_ref, out_ref.at[i, pl.ds(s8, _WE), :], sem_ref.at[0])
    cp_out.start()
    cp_out.wait()


def kernel(words, sent_id, heads_ids, emb,
           W1, b1, W2, b2, W3, b3, c1w, c1b, c2w, c2b):
    n_words, in_size = words.shape
    n_heads = heads_ids.shape[0]

    heads = heads_ids.astype(jnp.int32)
    sid_h = jnp.take(sent_id, heads, axis=0)
    starts = jnp.searchsorted(sent_id, sid_h, side="left").astype(jnp.int32)
    ends = jnp.searchsorted(sent_id, sid_h, side="right").astype(jnp.int32)
    lens = ends - starts
    ml = jnp.max(lens)[None].astype(jnp.int32)

    words_pad = jnp.concatenate(
        [words, jnp.zeros((_WE, in_size), words.dtype)], axis=0)
    words_pad_shape = words_pad.shape

    w1h = W1[:, :in_size].T
    w1c = W1[:, in_size:2 * in_size].T
    w1e = W1[:, 2 * in_size:].T
    c1wT = jnp.transpose(c1w, (2, 1, 0))
    c2wT = jnp.transpose(c2w, (2, 1, 0))

    grid_spec = pltpu.PrefetchScalarGridSpec(
        num_scalar_prefetch=4,
        grid=(n_heads,),
        in_specs=[
            pl.BlockSpec(words_pad_shape, lambda i, *_: (0, 0)),
            pl.BlockSpec(emb.shape, lambda i, *_: (0, 0)),
            pl.BlockSpec(w1h.shape, lambda i, *_: (0, 0)),
            pl.BlockSpec(w1c.shape, lambda i, *_: (0, 0)),
            pl.BlockSpec(w1e.shape, lambda i, *_: (0, 0)),
            pl.BlockSpec((1, b1.shape[0]), lambda i, *_: (0, 0)),
            pl.BlockSpec((W2.shape[1], W2.shape[0]), lambda i, *_: (0, 0)),
            pl.BlockSpec((1, b2.shape[0]), lambda i, *_: (0, 0)),
            pl.BlockSpec((W3.shape[1], W3.shape[0]), lambda i, *_: (0, 0)),
            pl.BlockSpec((1, b3.shape[0]), lambda i, *_: (0, 0)),
            pl.BlockSpec(c1wT.shape, lambda i, *_: (0, 0, 0)),
            pl.BlockSpec((1, c1b.shape[0]), lambda i, *_: (0, 0)),
            pl.BlockSpec(c2wT.shape, lambda i, *_: (0, 0, 0)),
            pl.BlockSpec((1, c2b.shape[0]), lambda i, *_: (0, 0)),
            pl.BlockSpec(memory_space=pl.ANY),
        ],
        out_specs=pl.BlockSpec(memory_space=pl.ANY),
        scratch_shapes=[
            pltpu.VMEM((_WE, 2), jnp.float32),
            pltpu.SemaphoreType.DMA((1,)),
        ],
    )

    init = jnp.full((n_heads, n_words + _WE, 2), _NEG_INF, jnp.float32)
    out_pad = pl.pallas_call(
        _tc_kernel,
        grid_spec=grid_spec,
        out_shape=jax.ShapeDtypeStruct((n_heads, n_words + _WE, 2),
                                       jnp.float32),
        input_output_aliases={18: 0},
    )(starts, lens, heads, ml,
      words_pad, emb,
      w1h, w1c, w1e, b1[None, :],
      W2.T, b2[None, :], W3.T, b3[None, :],
      c1wT, c1b[None, :], c2wT, c2b[None, :], init)

    return out_pad[:, :n_words]


# speedup vs baseline: 27.8755x; 1.2180x over previous
import jax
import jax.numpy as jnp
from jax.experimental import pallas as pl
from jax.experimental.pallas import tpu as pltpu

_WE = 128
_B = 8
_NEG_INF = float("-inf")


def _dot(a, b):
    return jax.lax.dot_general(a, b, (((1,), (0,)), ((), ())),
                               preferred_element_type=jnp.float32)


def _tc_kernel(starts_ref, lens_ref, heads_ref, ml_ref,
               words_ref, emb_ref,
               w1h_ref, w1c_ref, w1e_ref, b1_ref,
               w2_ref, b2_ref, w3_ref, b3_ref,
               c1w_ref, c1b_ref, c2w_ref, c2b_ref, init_ref,
               out_ref,
               owin_ref, sem_ref):
    i = pl.program_id(0)
    ml = ml_ref[0]
    in_size = words_ref.shape[1]

    t = jax.lax.broadcasted_iota(jnp.int32, (_WE, 1), 0)
    t8 = jax.lax.broadcasted_iota(jnp.int32, (8, 1), 0)

    wins, hrows, rels, spans, inlens, s8s, js = [], [], [], [], [], [], []
    for b in range(_B):
        j = i * _B + b
        s = starts_ref[j]
        L = lens_ref[j]
        hd = heads_ref[j]
        s8 = pl.multiple_of((s // 8) * 8, 8)
        d = s - s8
        hd8 = pl.multiple_of((hd // 8) * 8, 8)
        wins.append(words_ref[pl.ds(s8, _WE), :])
        hblk = words_ref[pl.ds(hd8, 8), :]
        hrow = jnp.sum(jnp.where(t8 == hd - hd8, hblk, 0.0),
                       axis=0, keepdims=True)
        hrows.append(hrow)
        rels.append(hd - (s8 + t))
        spans.append((t >= d) & (t < d + L))
        inlens.append((t >= d) & (t < d + ml))
        s8s.append(s8)
        js.append(j)

    win = jnp.concatenate(wins, axis=0)
    rel = jnp.concatenate(rels, axis=0)
    in_span = jnp.concatenate(spans, axis=0)
    in_len = jnp.concatenate(inlens, axis=0)

    hstack = jnp.concatenate(hrows, axis=0)
    ph = _dot(hstack, w1h_ref[...])
    phx = jnp.concatenate(
        [jnp.broadcast_to(ph[b:b + 1], (_WE, ph.shape[1]))
         for b in range(_B)], axis=0)

    eid = rel + 63
    eid = jnp.where((eid < 0) | (eid > 126), 127, eid)
    lane = jax.lax.broadcasted_iota(jnp.int32, (_B * _WE, 128), 1)
    onehot = (lane == eid).astype(jnp.float32)
    embrows = _dot(onehot, emb_ref[...])

    pre = (_dot(win, w1c_ref[...])
           + _dot(embrows, w1e_ref[...])
           + phx)
    pre = jnp.where(in_span, pre, 0.0)
    h = jnp.maximum(pre + b1_ref[...], 0.0)
    h = jnp.maximum(_dot(h, w2_ref[...]) + b2_ref[...], 0.0)
    h = _dot(h, w3_ref[...]) + b3_ref[...]
    hm = jnp.where(in_len, h, 0.0)

    z64 = jnp.zeros((1, 64), jnp.float32)
    hprev = jnp.concatenate([z64, hm[:-1]], axis=0)
    hnext = jnp.concatenate([hm[1:], z64], axis=0)
    g = (_dot(hprev, c1w_ref[0]) + _dot(hm, c1w_ref[1])
         + _dot(hnext, c1w_ref[2]) + c1b_ref[...])
    gm = jnp.where(in_len, g, 0.0)

    z4 = jnp.zeros((1, 4), jnp.float32)
    gprev = jnp.concatenate([z4, gm[:-1]], axis=0)
    gnext = jnp.concatenate([gm[1:], z4], axis=0)
    r = (_dot(gprev, c2w_ref[0]) + _dot(gm, c2w_ref[1])
         + _dot(gnext, c2w_ref[2]) + c2b_ref[...])

    lane2 = jax.lax.broadcasted_iota(jnp.int32, (_B * _WE, 2), 1)
    sgn = 1 - 2 * lane2
    valid = ((rel * sgn) >= 0) & in_span
    owin_ref[...] = jnp.where(valid, r, _NEG_INF)

    cps = []
    for b in range(_B):
        cp = pltpu.make_async_copy(
            owin_ref.at[pl.ds(b * _WE, _WE), :],
            out_ref.at[js[b], pl.ds(s8s[b], _WE), :],
            sem_ref.at[0])
        cp.start()
        cps.append(cp)
    for cp in cps:
        cp.wait()


def kernel(words, sent_id, heads_ids, emb,
           W1, b1, W2, b2, W3, b3, c1w, c1b, c2w, c2b):
    n_words, in_size = words.shape
    n_heads = heads_ids.shape[0]

    heads = heads_ids.astype(jnp.int32)
    sid_h = jnp.take(sent_id, heads, axis=0)
    starts = jnp.searchsorted(sent_id, sid_h, side="left").astype(jnp.int32)
    ends = jnp.searchsorted(sent_id, sid_h, side="right").astype(jnp.int32)
    lens = ends - starts
    ml = jnp.max(lens)[None].astype(jnp.int32)

    words_pad = jnp.concatenate(
        [words, jnp.zeros((_WE, in_size), words.dtype)], axis=0)
    words_pad_shape = words_pad.shape

    w1h = W1[:, :in_size].T
    w1c = W1[:, in_size:2 * in_size].T
    w1e = W1[:, 2 * in_size:].T
    c1wT = jnp.transpose(c1w, (2, 1, 0))
    c2wT = jnp.transpose(c2w, (2, 1, 0))

    grid_spec = pltpu.PrefetchScalarGridSpec(
        num_scalar_prefetch=4,
        grid=(n_heads // _B,),
        in_specs=[
            pl.BlockSpec(words_pad_shape, lambda i, *_: (0, 0)),
            pl.BlockSpec(emb.shape, lambda i, *_: (0, 0)),
            pl.BlockSpec(w1h.shape, lambda i, *_: (0, 0)),
            pl.BlockSpec(w1c.shape, lambda i, *_: (0, 0)),
            pl.BlockSpec(w1e.shape, lambda i, *_: (0, 0)),
            pl.BlockSpec((1, b1.shape[0]), lambda i, *_: (0, 0)),
            pl.BlockSpec((W2.shape[1], W2.shape[0]), lambda i, *_: (0, 0)),
            pl.BlockSpec((1, b2.shape[0]), lambda i, *_: (0, 0)),
            pl.BlockSpec((W3.shape[1], W3.shape[0]), lambda i, *_: (0, 0)),
            pl.BlockSpec((1, b3.shape[0]), lambda i, *_: (0, 0)),
            pl.BlockSpec(c1wT.shape, lambda i, *_: (0, 0, 0)),
            pl.BlockSpec((1, c1b.shape[0]), lambda i, *_: (0, 0)),
            pl.BlockSpec(c2wT.shape, lambda i, *_: (0, 0, 0)),
            pl.BlockSpec((1, c2b.shape[0]), lambda i, *_: (0, 0)),
            pl.BlockSpec(memory_space=pl.ANY),
        ],
        out_specs=pl.BlockSpec(memory_space=pl.ANY),
        scratch_shapes=[
            pltpu.VMEM((_B * _WE, 2), jnp.float32),
            pltpu.SemaphoreType.DMA((1,)),
        ],
    )

    init = jnp.full((n_heads, n_words + _WE, 2), _NEG_INF, jnp.float32)
    out_pad = pl.pallas_call(
        _tc_kernel,
        grid_spec=grid_spec,
        out_shape=jax.ShapeDtypeStruct((n_heads, n_words + _WE, 2),
                                       jnp.float32),
        input_output_aliases={18: 0},
    )(starts, lens, heads, ml,
      words_pad, emb,
      w1h, w1c, w1e, b1[None, :],
      W2.T, b2[None, :], W3.T, b3[None, :],
      c1wT, c1b[None, :], c2wT, c2b[None, :], init)

    return out_pad[:, :n_words]


# speedup vs baseline: 41.1656x; 1.4768x over previous
import jax
import jax.numpy as jnp
from jax.experimental import pallas as pl
from jax.experimental.pallas import tpu as pltpu

_WE = 128
_B = 8
_OW = 2 * _WE
_NEG_INF = float("-inf")


def _dot(a, b):
    return jax.lax.dot_general(a, b, (((1,), (0,)), ((), ())),
                               preferred_element_type=jnp.float32)


def _tc_kernel(starts_ref, lens_ref, heads_ref, ml_ref,
               words_hbm_ref, emb_ref,
               w1h_ref, w1c_ref, w1e_ref, b1_ref,
               w2_ref, b2_ref, w3_ref, b3_ref,
               c1w_ref, c1b_ref, c2w_ref, c2b_ref,
               out_ref,
               wpad_ref, neg_ref, owin_ref, sem_ref):
    i = pl.program_id(0)
    ml = ml_ref[0]
    n_words = words_hbm_ref.shape[0]

    @pl.when(i == 0)
    def _():
        neg_ref[...] = jnp.full(neg_ref.shape, _NEG_INF, jnp.float32)
        cp = pltpu.make_async_copy(
            words_hbm_ref, wpad_ref.at[pl.ds(0, n_words), :], sem_ref.at[2])
        cp.start()
        cp.wait()

    fills = []
    for b in range(_B):
        cpf = pltpu.make_async_copy(
            neg_ref, out_ref.at[i * _B + b], sem_ref.at[1])
        cpf.start()
        fills.append(cpf)

    t = jax.lax.broadcasted_iota(jnp.int32, (_WE, 1), 0)
    t8 = jax.lax.broadcasted_iota(jnp.int32, (8, 1), 0)

    wins, hrows, rels, spans, inlens, s8s, js = [], [], [], [], [], [], []
    for b in range(_B):
        j = i * _B + b
        s = starts_ref[j]
        L = lens_ref[j]
        hd = heads_ref[j]
        s8 = pl.multiple_of((s // 8) * 8, 8)
        d = s - s8
        hd8 = pl.multiple_of((hd // 8) * 8, 8)
        wins.append(wpad_ref[pl.ds(s8, _WE), :])
        hblk = wpad_ref[pl.ds(hd8, 8), :]
        hrow = jnp.sum(jnp.where(t8 == hd - hd8, hblk, 0.0),
                       axis=0, keepdims=True)
        hrows.append(hrow)
        rels.append(hd - (s8 + t))
        spans.append((t >= d) & (t < d + L))
        inlens.append((t >= d) & (t < d + ml))
        s8s.append(s8)
        js.append(j)

    win = jnp.concatenate(wins, axis=0)
    rel = jnp.concatenate(rels, axis=0)
    in_span = jnp.concatenate(spans, axis=0)
    in_len = jnp.concatenate(inlens, axis=0)

    hstack = jnp.concatenate(hrows, axis=0)
    ph = _dot(hstack, w1h_ref[...])
    phx = jnp.concatenate(
        [jnp.broadcast_to(ph[b:b + 1], (_WE, ph.shape[1]))
         for b in range(_B)], axis=0)

    eid = rel + 63
    eid = jnp.where((eid < 0) | (eid > 126), 127, eid)
    lane = jax.lax.broadcasted_iota(jnp.int32, (_B * _WE, 128), 1)
    onehot = (lane == eid).astype(jnp.float32)
    embrows = _dot(onehot, emb_ref[...])

    pre = (_dot(win, w1c_ref[...])
           + _dot(embrows, w1e_ref[...])
           + phx)
    pre = jnp.where(in_span, pre, 0.0)
    h = jnp.maximum(pre + b1_ref[...], 0.0)
    h = jnp.maximum(_dot(h, w2_ref[...]) + b2_ref[...], 0.0)
    h = _dot(h, w3_ref[...]) + b3_ref[...]
    hm = jnp.where(in_len, h, 0.0)

    z64 = jnp.zeros((1, 64), jnp.float32)
    hprev = jnp.concatenate([z64, hm[:-1]], axis=0)
    hnext = jnp.concatenate([hm[1:], z64], axis=0)
    g = (_dot(hprev, c1w_ref[0]) + _dot(hm, c1w_ref[1])
         + _dot(hnext, c1w_ref[2]) + c1b_ref[...])
    gm = jnp.where(in_len, g, 0.0)

    z4 = jnp.zeros((1, 4), jnp.float32)
    gprev = jnp.concatenate([z4, gm[:-1]], axis=0)
    gnext = jnp.concatenate([gm[1:], z4], axis=0)
    r = (_dot(gprev, c2w_ref[0]) + _dot(gm, c2w_ref[1])
         + _dot(gnext, c2w_ref[2]) + c2b_ref[...])

    lane2 = jax.lax.broadcasted_iota(jnp.int32, (_B * _WE, 2), 1)
    sgn = 1 - 2 * lane2
    valid = ((rel * sgn) >= 0) & in_span
    rv = jnp.where(valid, r, _NEG_INF)

    ninf = jnp.full((_WE, 2), _NEG_INF, jnp.float32)
    owin_ref[...] = jnp.concatenate(
        sum([[ninf, rv[b * _WE:(b + 1) * _WE]] for b in range(_B)], []),
        axis=0)

    for cpf in fills:
        cpf.wait()

    cps = []
    for b in range(_B):
        s8c = jnp.minimum(s8s[b], n_words - _WE)
        shift = s8s[b] - s8c
        src0 = pl.multiple_of(b * _OW + _WE - shift, 8)
        cp = pltpu.make_async_copy(
            owin_ref.at[pl.ds(src0, _WE), :],
            out_ref.at[js[b], pl.ds(pl.multiple_of(s8c, 8), _WE), :],
            sem_ref.at[0])
        cp.start()
        cps.append(cp)
    for cp in cps:
        cp.wait()


def kernel(words, sent_id, heads_ids, emb,
           W1, b1, W2, b2, W3, b3, c1w, c1b, c2w, c2b):
    n_words, in_size = words.shape
    n_heads = heads_ids.shape[0]

    heads = heads_ids.astype(jnp.int32)
    sid_h = jnp.take(sent_id, heads, axis=0)
    starts = jnp.searchsorted(sent_id, sid_h, side="left").astype(jnp.int32)
    ends = jnp.searchsorted(sent_id, sid_h, side="right").astype(jnp.int32)
    lens = ends - starts
    ml = jnp.max(lens)[None].astype(jnp.int32)

    w1h = W1[:, :in_size].T
    w1c = W1[:, in_size:2 * in_size].T
    w1e = W1[:, 2 * in_size:].T
    c1wT = jnp.transpose(c1w, (2, 1, 0))
    c2wT = jnp.transpose(c2w, (2, 1, 0))

    grid_spec = pltpu.PrefetchScalarGridSpec(
        num_scalar_prefetch=4,
        grid=(n_heads // _B,),
        in_specs=[
            pl.BlockSpec(memory_space=pl.ANY),
            pl.BlockSpec(emb.shape, lambda i, *_: (0, 0)),
            pl.BlockSpec(w1h.shape, lambda i, *_: (0, 0)),
            pl.BlockSpec(w1c.shape, lambda i, *_: (0, 0)),
            pl.BlockSpec(w1e.shape, lambda i, *_: (0, 0)),
            pl.BlockSpec((1, b1.shape[0]), lambda i, *_: (0, 0)),
            pl.BlockSpec((W2.shape[1], W2.shape[0]), lambda i, *_: (0, 0)),
            pl.BlockSpec((1, b2.shape[0]), lambda i, *_: (0, 0)),
            pl.BlockSpec((W3.shape[1], W3.shape[0]), lambda i, *_: (0, 0)),
            pl.BlockSpec((1, b3.shape[0]), lambda i, *_: (0, 0)),
            pl.BlockSpec(c1wT.shape, lambda i, *_: (0, 0, 0)),
            pl.BlockSpec((1, c1b.shape[0]), lambda i, *_: (0, 0)),
            pl.BlockSpec(c2wT.shape, lambda i, *_: (0, 0, 0)),
            pl.BlockSpec((1, c2b.shape[0]), lambda i, *_: (0, 0)),
        ],
        out_specs=pl.BlockSpec(memory_space=pl.ANY),
        scratch_shapes=[
            pltpu.VMEM((n_words + _WE, in_size), jnp.float32),
            pltpu.VMEM((n_words, 2), jnp.float32),
            pltpu.VMEM((_B * _OW, 2), jnp.float32),
            pltpu.SemaphoreType.DMA((3,)),
        ],
    )

    return pl.pallas_call(
        _tc_kernel,
        grid_spec=grid_spec,
        out_shape=jax.ShapeDtypeStruct((n_heads, n_words, 2), jnp.float32),
    )(starts, lens, heads, ml,
      words, emb,
      w1h, w1c, w1e, b1[None, :],
      W2.T, b2[None, :], W3.T, b3[None, :],
      c1wT, c1b[None, :], c2wT, c2b[None, :])
